# baseline (speedup 1.0000x reference)
import jax
import jax.numpy as jnp
from jax.experimental import pallas as pl

_TN = 256
_N = 1024


def _body(v5_ref, x1_ref, v1_ref, v7r_ref, x10_ref, x11_ref, x12_ref):
    n = pl.program_id(1)
    x6v = jnp.max(v5_ref[0], axis=0)
    x1t = x1_ref[0]
    v1t = v1_ref[0]
    v7t = v7r_ref[0]
    x9 = jax.nn.sigmoid(x1t + v7t)
    p = x9 * v1t
    top_a = p * x1t
    x6col = x6v[:, None]
    top_b = p * x6col
    x6b = jnp.broadcast_to(x6col, x1t.shape)
    x10_ref[0, 0] = x1t
    x10_ref[0, 1] = x6b
    x12_ref[0, 0] = x1t + top_a
    x12_ref[0, 1] = x6b + top_b
    x11_ref[0, :, pl.ds(n * _TN, _TN)] = top_a.T
    x11_ref[0, :, pl.ds(_N + n * _TN, _TN)] = top_b.T


def kernel(x1, v1, v5, v6r, v7r):
    del v6r
    B, N, D = x1.shape

    strip = pl.BlockSpec((1, _TN, D), lambda b, n: (b, n, 0))
    x10, x11, x12 = pl.pallas_call(
        _body,
        grid=(B, N // _TN),
        in_specs=[
            pl.BlockSpec((1, N, _TN), lambda b, n: (b, 0, n)),
            strip,
            strip,
            strip,
        ],
        out_specs=[
            pl.BlockSpec((1, 2, _TN, D), lambda b, n: (b, 0, n, 0)),
            pl.BlockSpec((1, D, 2 * N), lambda b, n: (b, 0, 0)),
            pl.BlockSpec((1, 2, _TN, D), lambda b, n: (b, 0, n, 0)),
        ],
        out_shape=[
            jax.ShapeDtypeStruct((B, 2, N, D), jnp.float32),
            jax.ShapeDtypeStruct((B, D, 2 * N), jnp.float32),
            jax.ShapeDtypeStruct((B, 2, N, D), jnp.float32),
        ],
    )(v5, x1, v1, v7r)
    return (
        x10.reshape(B, 2 * N, D),
        x11,
        x12.reshape(B, 2 * N, D),
    )


# speedup vs baseline: 1.0473x; 1.0473x over previous
import jax
import jax.numpy as jnp
from jax.experimental import pallas as pl

_TN = 512
_N = 1024


def _body(v5_ref, x1_ref, v1_ref, v7r_ref, x10_ref, x11_ref, x12_ref):
    n = pl.program_id(1)
    x6v = jnp.max(v5_ref[0], axis=0)
    x1t = x1_ref[0]
    v1t = v1_ref[0]
    v7t = v7r_ref[0]
    x9 = jax.nn.sigmoid(x1t + v7t)
    p = x9 * v1t
    top_a = p * x1t
    x6col = x6v[:, None]
    top_b = p * x6col
    x6b = jnp.broadcast_to(x6col, x1t.shape)
    x10_ref[0, 0] = x1t
    x10_ref[0, 1] = x6b
    x12_ref[0, 0] = x1t + top_a
    x12_ref[0, 1] = x6b + top_b
    x11_ref[0, :, pl.ds(n * _TN, _TN)] = top_a.T
    x11_ref[0, :, pl.ds(_N + n * _TN, _TN)] = top_b.T


def kernel(x1, v1, v5, v6r, v7r):
    del v6r
    B, N, D = x1.shape

    strip = pl.BlockSpec((1, _TN, D), lambda b, n: (b, n, 0))
    x10, x11, x12 = pl.pallas_call(
        _body,
        grid=(B, N // _TN),
        in_specs=[
            pl.BlockSpec((1, N, _TN), lambda b, n: (b, 0, n)),
            strip,
            strip,
            strip,
        ],
        out_specs=[
            pl.BlockSpec((1, 2, _TN, D), lambda b, n: (b, 0, n, 0)),
            pl.BlockSpec((1, D, 2 * N), lambda b, n: (b, 0, 0)),
            pl.BlockSpec((1, 2, _TN, D), lambda b, n: (b, 0, n, 0)),
        ],
        out_shape=[
            jax.ShapeDtypeStruct((B, 2, N, D), jnp.float32),
            jax.ShapeDtypeStruct((B, D, 2 * N), jnp.float32),
            jax.ShapeDtypeStruct((B, 2, N, D), jnp.float32),
        ],
    )(v5, x1, v1, v7r)
    return (
        x10.reshape(B, 2 * N, D),
        x11,
        x12.reshape(B, 2 * N, D),
    )
